# Initial kernel scaffold; baseline (speedup 1.0000x reference)
import jax
import jax.numpy as jnp
from jax.experimental import pallas as pl


def kernel(x, routing_weights, W1, b1, W2, b2):
    raise NotImplementedError("write your pallas kernel here")


# speedup vs baseline: 1.1873x; 1.1873x over previous
import functools

import jax
import jax.numpy as jnp
from jax.experimental import pallas as pl
from jax.experimental.pallas import tpu as pltpu

NUM_EXPERTS = 8
TOP_K = 2
D_MODEL = 1024
EXPERT_DIM = 2048
FJ = 512
NJ = EXPERT_DIM // FJ


def _moe_body(x_ref, rw_ref, w1_ref, b1_ref, w2_ref, b2_ref, out_ref, ew_ref):
    e = pl.program_id(0)
    j = pl.program_id(1)
    T = x_ref.shape[0]

    @pl.when((e == 0) & (j == 0))
    def _routing():
        logits = jnp.dot(x_ref[...], rw_ref[...],
                         preferred_element_type=jnp.float32)
        m = jnp.max(logits, axis=-1, keepdims=True)
        p = jnp.exp(logits - m)
        p = p / jnp.sum(p, axis=-1, keepdims=True)
        idx = jax.lax.broadcasted_iota(jnp.int32, (T, NUM_EXPERTS), 1)
        v1 = jnp.max(p, axis=-1, keepdims=True)
        i1 = jnp.min(jnp.where(p == v1, idx, NUM_EXPERTS), axis=-1, keepdims=True)
        oh1 = idx == i1
        p2 = jnp.where(oh1, -1.0, p)
        v2 = jnp.max(p2, axis=-1, keepdims=True)
        i2 = jnp.min(jnp.where(p2 == v2, idx, NUM_EXPERTS), axis=-1, keepdims=True)
        oh2 = idx == i2
        ew_ref[...] = jnp.where(oh1 | oh2, p, 0.0)
        out_ref[...] = jnp.zeros_like(out_ref)

    idx = jax.lax.broadcasted_iota(jnp.int32, (T, NUM_EXPERTS), 1)
    w = jnp.sum(jnp.where(idx == e, ew_ref[...], 0.0), axis=-1,
                keepdims=True)
    h = jnp.dot(x_ref[...], w1_ref[0], preferred_element_type=jnp.float32)
    h = jax.nn.gelu(h + b1_ref[0])
    contrib = jnp.dot(h, w2_ref[0], preferred_element_type=jnp.float32)

    acc = w * contrib

    @pl.when(j == 0)
    def _bias2():
        out_ref[...] += w * b2_ref[0]

    out_ref[...] += acc


@jax.jit
def kernel(x, routing_weights, W1, b1, W2, b2):
    B, S, D = x.shape
    T = B * S
    x2 = x.reshape(T, D)

    grid = (NUM_EXPERTS, NJ)
    out = pl.pallas_call(
        _moe_body,
        grid=grid,
        in_specs=[
            pl.BlockSpec((T, D), lambda e, j: (0, 0)),
            pl.BlockSpec((D, NUM_EXPERTS), lambda e, j: (0, 0)),
            pl.BlockSpec((1, D, FJ), lambda e, j: (e, 0, j)),
            pl.BlockSpec((1, 1, FJ), lambda e, j: (e, 0, j)),
            pl.BlockSpec((1, FJ, D), lambda e, j: (e, j, 0)),
            pl.BlockSpec((1, 1, D), lambda e, j: (e, 0, 0)),
        ],
        out_specs=pl.BlockSpec((T, D), lambda e, j: (0, 0)),
        out_shape=jax.ShapeDtypeStruct((T, D), jnp.float32),
        scratch_shapes=[pltpu.VMEM((T, NUM_EXPERTS), jnp.float32)],
    )(x2, routing_weights, W1, b1.reshape(NUM_EXPERTS, 1, EXPERT_DIM),
      W2, b2.reshape(NUM_EXPERTS, 1, D_MODEL))
    return out.reshape(B, S, D)
